# baseline (speedup 1.0000x reference)
import functools

import jax
import jax.numpy as jnp
from jax import lax
from jax.experimental import pallas as pl
from jax.experimental.pallas import tpu as pltpu
from jax.experimental.pallas import tpu_sc as plsc

N = 10000
E = 160000
F_IN = 128
F_HID = 64
NC = 2
NS = 16
NW = NC * NS
EPAD = 160256
CPW = EPAD // NW
NCHUNK = CPW // 16
NROWS = 40
ND = 10240
STRIPE = ND // NS
PAD_DST = 10200


def _lrelu(x):
    return jnp.where(x >= 0, x, 0.01 * x)


_BLK = 1000
_NBLK = N // _BLK


def _tc_in_body(x_ref, wfcT_ref, wdiffT_ref, att_ref, z_ref, hd_ref, s_ref,
                smx_ref, smn_ref):
    i = pl.program_id(0)
    xb = x_ref[...]
    z_ref[...] = jnp.dot(xb, wfcT_ref[...], preferred_element_type=jnp.float32)
    hd = jnp.dot(xb, wdiffT_ref[...], preferred_element_type=jnp.float32)
    hd_ref[...] = hd
    s = jnp.dot(hd, att_ref[...], preferred_element_type=jnp.float32)
    s_ref[...] = s

    @pl.when(i == 0)
    def _():
        smx_ref[...] = jnp.full((1, 128), -jnp.inf, jnp.float32)
        smn_ref[...] = jnp.full((1, 128), jnp.inf, jnp.float32)

    smx_ref[...] = jnp.maximum(smx_ref[...], jnp.full((1, 128), jnp.max(s)))
    smn_ref[...] = jnp.minimum(smn_ref[...], jnp.full((1, 128), jnp.min(s)))


def _tc_input_layer(x, wfcT, wdiffT, att):
    fz = wfcT.shape[1]
    fh = wdiffT.shape[1]
    return pl.pallas_call(
        _tc_in_body,
        grid=(_NBLK,),
        in_specs=[
            pl.BlockSpec((_BLK, x.shape[1]), lambda i: (i, 0)),
            pl.BlockSpec(wfcT.shape, lambda i: (0, 0)),
            pl.BlockSpec(wdiffT.shape, lambda i: (0, 0)),
            pl.BlockSpec(att.shape, lambda i: (0, 0)),
        ],
        out_specs=[
            pl.BlockSpec((_BLK, fz), lambda i: (i, 0)),
            pl.BlockSpec((_BLK, fh), lambda i: (i, 0)),
            pl.BlockSpec((_BLK, 1), lambda i: (i, 0)),
            pl.BlockSpec((1, 128), lambda i: (0, 0)),
            pl.BlockSpec((1, 128), lambda i: (0, 0)),
        ],
        out_shape=[
            jax.ShapeDtypeStruct((N, fz), jnp.float32),
            jax.ShapeDtypeStruct((N, fh), jnp.float32),
            jax.ShapeDtypeStruct((N, 1), jnp.float32),
            jax.ShapeDtypeStruct((1, 128), jnp.float32),
            jax.ShapeDtypeStruct((1, 128), jnp.float32),
        ],
    )(x, wfcT, wdiffT, att)


def _tc_mid_body(z_ref, hd_ref, acc_ref, d0_ref, d1_ref, wfcT_ref, wdiffT_ref,
                 att_ref, z2_ref, hd2_ref, s2_ref, smx_ref, smn_ref):
    i = pl.program_id(0)
    fz = z_ref.shape[1]
    acc = acc_ref[0][:, :fz] + acc_ref[1][:, :fz]
    has = ((d0_ref[...] + d1_ref[...]) > 0).astype(jnp.float32)
    h = z_ref[...] + acc - hd_ref[...][:, :fz] * has
    h1 = jnp.where(h > 0, h, jnp.exp(h) - 1.0)
    z2_ref[...] = jnp.dot(h1, wfcT_ref[...], preferred_element_type=jnp.float32)
    hd2 = jnp.dot(h1, wdiffT_ref[...], preferred_element_type=jnp.float32)
    hd2_ref[...] = hd2
    s2 = jnp.dot(hd2, att_ref[...], preferred_element_type=jnp.float32)
    s2_ref[...] = s2

    @pl.when(i == 0)
    def _():
        smx_ref[...] = jnp.full((1, 128), -jnp.inf, jnp.float32)
        smn_ref[...] = jnp.full((1, 128), jnp.inf, jnp.float32)

    smx_ref[...] = jnp.maximum(smx_ref[...], jnp.full((1, 128), jnp.max(s2)))
    smn_ref[...] = jnp.minimum(smn_ref[...], jnp.full((1, 128), jnp.min(s2)))


def _tc_mid_layer(z, hd, acc, d0, d1, wfcT, wdiffT, att):
    fi = z.shape[1]
    fw = hd.shape[1]
    fo = wfcT.shape[1]
    return pl.pallas_call(
        _tc_mid_body,
        grid=(_NBLK,),
        in_specs=[
            pl.BlockSpec((_BLK, fi), lambda i: (i, 0)),
            pl.BlockSpec((_BLK, fw), lambda i: (i, 0)),
            pl.BlockSpec((NC, _BLK, fw), lambda i: (0, i, 0)),
            pl.BlockSpec((_BLK, 1), lambda i: (i, 0)),
            pl.BlockSpec((_BLK, 1), lambda i: (i, 0)),
            pl.BlockSpec(wfcT.shape, lambda i: (0, 0)),
            pl.BlockSpec(wdiffT.shape, lambda i: (0, 0)),
            pl.BlockSpec(att.shape, lambda i: (0, 0)),
        ],
        out_specs=[
            pl.BlockSpec((_BLK, fo), lambda i: (i, 0)),
            pl.BlockSpec((_BLK, fo), lambda i: (i, 0)),
            pl.BlockSpec((_BLK, 1), lambda i: (i, 0)),
            pl.BlockSpec((1, 128), lambda i: (0, 0)),
            pl.BlockSpec((1, 128), lambda i: (0, 0)),
        ],
        out_shape=[
            jax.ShapeDtypeStruct((N, fo), jnp.float32),
            jax.ShapeDtypeStruct((N, fo), jnp.float32),
            jax.ShapeDtypeStruct((N, 1), jnp.float32),
            jax.ShapeDtypeStruct((1, 128), jnp.float32),
            jax.ShapeDtypeStruct((1, 128), jnp.float32),
        ],
    )(z, hd, acc, d0, d1, wfcT, wdiffT, att)


def _tc_out_body(z_ref, hd_ref, acc_ref, d0_ref, d1_ref, out_ref):
    acc = acc_ref[0] + acc_ref[1]
    has = ((d0_ref[...] + d1_ref[...]) > 0).astype(jnp.float32)
    out_ref[...] = z_ref[...] + acc - hd_ref[...] * has


def _tc_out_layer(z, hd, acc, d0, d1):
    fo = z.shape[1]
    return pl.pallas_call(
        _tc_out_body,
        grid=(_NBLK,),
        in_specs=[
            pl.BlockSpec((_BLK, fo), lambda i: (i, 0)),
            pl.BlockSpec((_BLK, fo), lambda i: (i, 0)),
            pl.BlockSpec((NC, _BLK, fo), lambda i: (0, i, 0)),
            pl.BlockSpec((_BLK, 1), lambda i: (i, 0)),
            pl.BlockSpec((_BLK, 1), lambda i: (i, 0)),
        ],
        out_specs=pl.BlockSpec((_BLK, fo), lambda i: (i, 0)),
        out_shape=jax.ShapeDtypeStruct((N, fo), jnp.float32),
    )(z, hd, acc, d0, d1)


_MESH = plsc.VectorSubcoreMesh(core_axis_name="c", subcore_axis_name="s")


@functools.partial(
    pl.kernel,
    out_type=jax.ShapeDtypeStruct((NC * ND,), jnp.float32),
    mesh=_MESH,
    scratch_types=[
        pltpu.VMEM((CPW,), jnp.int32),
        pltpu.VMEM((CPW,), jnp.int32),
        pltpu.VMEM((ND,), jnp.float32),
        pltpu.VMEM((16,), jnp.float32),
        pltpu.VMEM((NROWS, 128), jnp.float32),
        pltpu.VMEM((NROWS, 128), jnp.int32),
        pltpu.VMEM_SHARED((ND,), jnp.float32),
    ],
    compiler_params=pltpu.CompilerParams(needs_layout_passes=False),
)
def _sc_pass_a(flat_hbm, prev_hbm, s_hbm, m_hbm, z640_hbm, den_hbm,
               flat_v, prev_v, s_v, m_v, p_m, dst_m, den_sh):
    c = lax.axis_index("c")
    sid = lax.axis_index("s")
    wid = c * NS + sid
    base = wid * CPW
    pltpu.sync_copy(flat_hbm.at[pl.ds(base, CPW)], flat_v)
    pltpu.sync_copy(prev_hbm.at[pl.ds(base, CPW)], prev_v)
    pltpu.sync_copy(s_hbm, s_v)
    pltpu.sync_copy(m_hbm, m_v)
    pltpu.sync_copy(z640_hbm, den_sh.at[pl.ds(sid * STRIPE, STRIPE)])
    plsc.subcore_barrier()

    mvec = m_v[...]
    pad_dst = jnp.full((16,), PAD_DST, jnp.int32)
    zero16 = jnp.zeros((16,), jnp.float32)
    slope16 = jnp.full((16,), 0.01, jnp.float32)
    nvec = jnp.full((16,), N, jnp.int32)
    for k in range(8):
        p_m[NROWS - 1, pl.ds(k * 16, 16)] = zero16
        dst_m[NROWS - 1, pl.ds(k * 16, 16)] = pad_dst

    def chunk(i, row, col):
        idx = pl.ds(i * 16, 16)
        f = flat_v[idx]
        fp = prev_v[idx]
        dstv = lax.div(f, nvec)
        srcv = f - dstv * nvec
        ssrc = plsc.load_gather(s_v, [srcv])
        sdst = plsc.load_gather(s_v, [dstv])
        t = ssrc - sdst
        e = jnp.where(t >= zero16, t, slope16 * t)
        p = jnp.where(f != fp, jnp.exp(e - mvec), zero16)
        p_m[row, pl.ds(col * 16, 16)] = p
        dst_m[row, pl.ds(col * 16, 16)] = dstv

    def rowbody(j, carry):
        for k in range(8):
            chunk(j * 8 + k, j, k)
        return carry

    lax.fori_loop(0, NROWS - 1, rowbody, 0)
    chunk((NROWS - 1) * 8, NROWS - 1, 0)

    def scatter_row(j, carry):
        pltpu.sync_copy(p_m.at[j], den_sh.at[dst_m.at[j]], add=True)
        return carry

    lax.fori_loop(0, NROWS, scatter_row, 0)
    plsc.subcore_barrier()
    pltpu.sync_copy(den_sh.at[pl.ds(sid * STRIPE, STRIPE)],
                    den_hbm.at[pl.ds(c * ND + sid * STRIPE, STRIPE)])


def _make_sc_pass_b(F):
    @functools.partial(
        pl.kernel,
        out_type=jax.ShapeDtypeStruct((NC * ND, F), jnp.float32),
        mesh=_MESH,
        scratch_types=[
            pltpu.VMEM((CPW,), jnp.int32),
            pltpu.VMEM((CPW,), jnp.int32),
            pltpu.VMEM((ND,), jnp.float32),
            pltpu.VMEM((16,), jnp.float32),
            pltpu.VMEM((ND,), jnp.float32),
            pltpu.VMEM((ND,), jnp.float32),
            pltpu.VMEM((16, F), jnp.float32),
            pltpu.VMEM_SHARED((ND, F), jnp.float32),
            pltpu.SemaphoreType.DMA,
        ],
        compiler_params=pltpu.CompilerParams(needs_layout_passes=False),
    )
    def sc_pass_b(flat_hbm, prev_hbm, s_hbm, m_hbm, den_hbm, hd_hbm, zrows_hbm,
                  acc_hbm, flat_v, prev_v, s_v, m_v, den_v, dent_v, rows_v,
                  acc_sh, sem):
        c = lax.axis_index("c")
        sid = lax.axis_index("s")
        wid = c * NS + sid
        base = wid * CPW
        pltpu.sync_copy(flat_hbm.at[pl.ds(base, CPW)], flat_v)
        pltpu.sync_copy(prev_hbm.at[pl.ds(base, CPW)], prev_v)
        pltpu.sync_copy(s_hbm, s_v)
        pltpu.sync_copy(m_hbm, m_v)
        pltpu.sync_copy(den_hbm.at[pl.ds(0, ND)], den_v)
        pltpu.sync_copy(den_hbm.at[pl.ds(ND, ND)], dent_v)

        def comb(i, carry):
            idx = pl.ds(i * 16, 16)
            den_v[idx] = den_v[idx] + dent_v[idx]
            return carry

        lax.fori_loop(0, ND // 16, comb, 0)
        pltpu.sync_copy(zrows_hbm, acc_sh.at[pl.ds(sid * STRIPE, STRIPE)])
        plsc.subcore_barrier()

        mvec = m_v[...]
        zero16 = jnp.zeros((16,), jnp.float32)
        slope16 = jnp.full((16,), 0.01, jnp.float32)
        nvec = jnp.full((16,), N, jnp.int32)

        def chunk(i, carry):
            idx = pl.ds(i * 16, 16)
            f = flat_v[idx]
            fp = prev_v[idx]
            dstv = lax.div(f, nvec)
            srcv = f - dstv * nvec
            ssrc = plsc.load_gather(s_v, [srcv])
            sdst = plsc.load_gather(s_v, [dstv])
            t = ssrc - sdst
            e = jnp.where(t >= zero16, t, slope16 * t)
            p = jnp.where(f != fp, jnp.exp(e - mvec), zero16)
            dv = plsc.load_gather(den_v, [dstv])
            a = p / dv
            pltpu.async_copy(hd_hbm.at[srcv], rows_v, sem).wait()
            dnums = lax.GatherDimensionNumbers(
                offset_dims=(), collapsed_slice_dims=(0,),
                start_index_map=(0,))
            for r in range(16):
                ar = lax.gather(
                    a, jnp.full((16, 1), r, jnp.int32), dnums, (1,),
                    mode=lax.GatherScatterMode.PROMISE_IN_BOUNDS)
                for cc in range(F // 16):
                    cs = pl.ds(cc * 16, 16)
                    rows_v[r, cs] = rows_v[r, cs] * ar
            pltpu.sync_copy(rows_v, acc_sh.at[dstv], add=True)
            return carry

        lax.fori_loop(0, NCHUNK, chunk, 0)
        plsc.subcore_barrier()
        pltpu.sync_copy(acc_sh.at[pl.ds(sid * STRIPE, STRIPE)],
                        acc_hbm.at[pl.ds(c * ND + sid * STRIPE, STRIPE)])

    return sc_pass_b


_sc_pass_b128 = _make_sc_pass_b(F_IN)


def kernel(x, edge_index, enc_fc_W, enc_diff_W, enc_att, att, dec_fc_W,
           dec_diff_W, dec_att):
    del att
    f32 = jnp.float32

    flat = jnp.sort(edge_index[1].astype(jnp.int32) * N
                    + edge_index[0].astype(jnp.int32))
    flat_p = jnp.concatenate(
        [flat, jnp.full((EPAD - E,), PAD_DST * N, jnp.int32)])
    flat_prev = jnp.concatenate([jnp.full((1,), -1, jnp.int32), flat_p[:-1]])
    z640 = jnp.zeros((STRIPE,), f32)
    zrows128 = jnp.zeros((STRIPE, F_IN), f32)

    wdiffT1 = jnp.concatenate(
        [enc_diff_W.T, jnp.zeros((F_IN, F_IN - F_HID), f32)], axis=1)
    att1 = jnp.concatenate(
        [enc_att, jnp.zeros((F_IN - F_HID, 1), f32)], axis=0)
    z1, hd1, s1, smx1, smn1 = _tc_input_layer(x, enc_fc_W.T, wdiffT1, att1)
    m1 = _lrelu(smx1[0, 0] - smn1[0, 0])
    m16_1 = jnp.full((16,), m1, f32)
    s1f = s1.reshape(-1)
    s1p = jnp.concatenate([s1f, jnp.broadcast_to(s1f[:1], (ND - N,))])
    den1 = _sc_pass_a(flat_p, flat_prev, s1p, m16_1, z640)
    acc1 = _sc_pass_b128(flat_p, flat_prev, s1p, m16_1, den1, hd1, zrows128)
    d1a = den1[:N].reshape(N, 1)
    d1b = den1[ND:ND + N].reshape(N, 1)

    z2, hd2, s2, smx2, smn2 = _tc_mid_layer(
        z1, hd1, acc1.reshape(NC, ND, F_IN), d1a, d1b,
        dec_fc_W.T, dec_diff_W.T, dec_att)
    m2 = _lrelu(smx2[0, 0] - smn2[0, 0])
    m16_2 = jnp.full((16,), m2, f32)
    s2f = s2.reshape(-1)
    s2p = jnp.concatenate([s2f, jnp.broadcast_to(s2f[:1], (ND - N,))])
    den2 = _sc_pass_a(flat_p, flat_prev, s2p, m16_2, z640)
    acc2 = _sc_pass_b128(flat_p, flat_prev, s2p, m16_2, den2, hd2, zrows128)
    d2a = den2[:N].reshape(N, 1)
    d2b = den2[ND:ND + N].reshape(N, 1)

    out = _tc_out_layer(z2, hd2, acc2.reshape(NC, ND, F_IN), d2a, d2b)
    return out


# speedup vs baseline: 1.1240x; 1.1240x over previous
import functools

import jax
import jax.numpy as jnp
from jax import lax
from jax.experimental import pallas as pl
from jax.experimental.pallas import tpu as pltpu
from jax.experimental.pallas import tpu_sc as plsc

N = 10000
E = 160000
F_IN = 128
F_HID = 64
NC = 2
NS = 16
NW = NC * NS
EPAD = 163840
CPW = EPAD // NW
NCHUNK = CPW // 16
NROWS = 40
GRP = 128
ND = 10240
STRIPE = ND // NS
PAD_DST = 10200


def _lrelu(x):
    return jnp.where(x >= 0, x, 0.01 * x)


_BLK = 1000
_NBLK = N // _BLK


def _tc_in_body(x_ref, wfcT_ref, wdiffT_ref, att_ref, z_ref, hd_ref, s_ref,
                smx_ref, smn_ref):
    i = pl.program_id(0)
    xb = x_ref[...]
    z_ref[...] = jnp.dot(xb, wfcT_ref[...], preferred_element_type=jnp.float32)
    hd = jnp.dot(xb, wdiffT_ref[...], preferred_element_type=jnp.float32)
    hd_ref[...] = hd
    s = jnp.dot(hd, att_ref[...], preferred_element_type=jnp.float32)
    s_ref[...] = s

    @pl.when(i == 0)
    def _():
        smx_ref[...] = jnp.full((1, 128), -jnp.inf, jnp.float32)
        smn_ref[...] = jnp.full((1, 128), jnp.inf, jnp.float32)

    smx_ref[...] = jnp.maximum(smx_ref[...], jnp.full((1, 128), jnp.max(s)))
    smn_ref[...] = jnp.minimum(smn_ref[...], jnp.full((1, 128), jnp.min(s)))


def _tc_input_layer(x, wfcT, wdiffT, att):
    fz = wfcT.shape[1]
    fh = wdiffT.shape[1]
    return pl.pallas_call(
        _tc_in_body,
        grid=(_NBLK,),
        in_specs=[
            pl.BlockSpec((_BLK, x.shape[1]), lambda i: (i, 0)),
            pl.BlockSpec(wfcT.shape, lambda i: (0, 0)),
            pl.BlockSpec(wdiffT.shape, lambda i: (0, 0)),
            pl.BlockSpec(att.shape, lambda i: (0, 0)),
        ],
        out_specs=[
            pl.BlockSpec((_BLK, fz), lambda i: (i, 0)),
            pl.BlockSpec((_BLK, fh), lambda i: (i, 0)),
            pl.BlockSpec((_BLK, 1), lambda i: (i, 0)),
            pl.BlockSpec((1, 128), lambda i: (0, 0)),
            pl.BlockSpec((1, 128), lambda i: (0, 0)),
        ],
        out_shape=[
            jax.ShapeDtypeStruct((N, fz), jnp.float32),
            jax.ShapeDtypeStruct((N, fh), jnp.float32),
            jax.ShapeDtypeStruct((N, 1), jnp.float32),
            jax.ShapeDtypeStruct((1, 128), jnp.float32),
            jax.ShapeDtypeStruct((1, 128), jnp.float32),
        ],
    )(x, wfcT, wdiffT, att)


def _tc_mid_body(z_ref, hd_ref, acc_ref, d0_ref, d1_ref, wfcT_ref, wdiffT_ref,
                 att_ref, z2_ref, hd2_ref, s2_ref, smx_ref, smn_ref):
    i = pl.program_id(0)
    fz = z_ref.shape[1]
    acc = acc_ref[0][:, :fz] + acc_ref[1][:, :fz]
    den = d0_ref[...] + d1_ref[...]
    hasmask = den > 0
    has = hasmask.astype(jnp.float32)
    dsafe = jnp.where(hasmask, den, 1.0)
    h = z_ref[...] + acc / dsafe - hd_ref[...][:, :fz] * has
    h1 = jnp.where(h > 0, h, jnp.exp(h) - 1.0)
    z2_ref[...] = jnp.dot(h1, wfcT_ref[...], preferred_element_type=jnp.float32)
    hd2 = jnp.dot(h1, wdiffT_ref[...], preferred_element_type=jnp.float32)
    hd2_ref[...] = hd2
    s2 = jnp.dot(hd2, att_ref[...], preferred_element_type=jnp.float32)
    s2_ref[...] = s2

    @pl.when(i == 0)
    def _():
        smx_ref[...] = jnp.full((1, 128), -jnp.inf, jnp.float32)
        smn_ref[...] = jnp.full((1, 128), jnp.inf, jnp.float32)

    smx_ref[...] = jnp.maximum(smx_ref[...], jnp.full((1, 128), jnp.max(s2)))
    smn_ref[...] = jnp.minimum(smn_ref[...], jnp.full((1, 128), jnp.min(s2)))


def _tc_mid_layer(z, hd, acc, d0, d1, wfcT, wdiffT, att):
    fi = z.shape[1]
    fw = hd.shape[1]
    fo = wfcT.shape[1]
    return pl.pallas_call(
        _tc_mid_body,
        grid=(_NBLK,),
        in_specs=[
            pl.BlockSpec((_BLK, fi), lambda i: (i, 0)),
            pl.BlockSpec((_BLK, fw), lambda i: (i, 0)),
            pl.BlockSpec((NC, _BLK, fw), lambda i: (0, i, 0)),
            pl.BlockSpec((_BLK, 1), lambda i: (i, 0)),
            pl.BlockSpec((_BLK, 1), lambda i: (i, 0)),
            pl.BlockSpec(wfcT.shape, lambda i: (0, 0)),
            pl.BlockSpec(wdiffT.shape, lambda i: (0, 0)),
            pl.BlockSpec(att.shape, lambda i: (0, 0)),
        ],
        out_specs=[
            pl.BlockSpec((_BLK, fo), lambda i: (i, 0)),
            pl.BlockSpec((_BLK, fo), lambda i: (i, 0)),
            pl.BlockSpec((_BLK, 1), lambda i: (i, 0)),
            pl.BlockSpec((1, 128), lambda i: (0, 0)),
            pl.BlockSpec((1, 128), lambda i: (0, 0)),
        ],
        out_shape=[
            jax.ShapeDtypeStruct((N, fo), jnp.float32),
            jax.ShapeDtypeStruct((N, fo), jnp.float32),
            jax.ShapeDtypeStruct((N, 1), jnp.float32),
            jax.ShapeDtypeStruct((1, 128), jnp.float32),
            jax.ShapeDtypeStruct((1, 128), jnp.float32),
        ],
    )(z, hd, acc, d0, d1, wfcT, wdiffT, att)


def _tc_out_body(z_ref, hd_ref, acc_ref, d0_ref, d1_ref, out_ref):
    acc = acc_ref[0] + acc_ref[1]
    den = d0_ref[...] + d1_ref[...]
    hasmask = den > 0
    has = hasmask.astype(jnp.float32)
    dsafe = jnp.where(hasmask, den, 1.0)
    out_ref[...] = z_ref[...] + acc / dsafe - hd_ref[...] * has


def _tc_out_layer(z, hd, acc, d0, d1):
    fo = z.shape[1]
    return pl.pallas_call(
        _tc_out_body,
        grid=(_NBLK,),
        in_specs=[
            pl.BlockSpec((_BLK, fo), lambda i: (i, 0)),
            pl.BlockSpec((_BLK, fo), lambda i: (i, 0)),
            pl.BlockSpec((NC, _BLK, fo), lambda i: (0, i, 0)),
            pl.BlockSpec((_BLK, 1), lambda i: (i, 0)),
            pl.BlockSpec((_BLK, 1), lambda i: (i, 0)),
        ],
        out_specs=pl.BlockSpec((_BLK, fo), lambda i: (i, 0)),
        out_shape=jax.ShapeDtypeStruct((N, fo), jnp.float32),
    )(z, hd, acc, d0, d1)


_MESH = plsc.VectorSubcoreMesh(core_axis_name="c", subcore_axis_name="s")


@functools.partial(
    pl.kernel,
    out_type=[
        jax.ShapeDtypeStruct((NC * ND,), jnp.float32),
        jax.ShapeDtypeStruct((NW * NROWS, GRP), jnp.float32),
        jax.ShapeDtypeStruct((NW * NROWS, GRP), jnp.int32),
        jax.ShapeDtypeStruct((NW * NROWS, GRP), jnp.int32),
    ],
    mesh=_MESH,
    scratch_types=[
        pltpu.VMEM((CPW,), jnp.int32),
        pltpu.VMEM((CPW,), jnp.int32),
        pltpu.VMEM((ND,), jnp.float32),
        pltpu.VMEM((16,), jnp.float32),
        pltpu.VMEM((NROWS, GRP), jnp.float32),
        pltpu.VMEM((NROWS, GRP), jnp.int32),
        pltpu.VMEM((NROWS, GRP), jnp.int32),
        pltpu.VMEM_SHARED((ND,), jnp.float32),
    ],
    compiler_params=pltpu.CompilerParams(needs_layout_passes=False),
)
def _sc_pass_a(flat_hbm, prev_hbm, s_hbm, m_hbm, z640_hbm,
               den_hbm, p_hbm, src_hbm, dst_hbm,
               flat_v, prev_v, s_v, m_v, p_m, src_m, dst_m, den_sh):
    c = lax.axis_index("c")
    sid = lax.axis_index("s")
    wid = c * NS + sid
    base = wid * CPW
    pltpu.sync_copy(flat_hbm.at[pl.ds(base, CPW)], flat_v)
    pltpu.sync_copy(prev_hbm.at[pl.ds(base, CPW)], prev_v)
    pltpu.sync_copy(s_hbm, s_v)
    pltpu.sync_copy(m_hbm, m_v)
    pltpu.sync_copy(z640_hbm, den_sh.at[pl.ds(sid * STRIPE, STRIPE)])
    plsc.subcore_barrier()

    mvec = m_v[...]
    zero16 = jnp.zeros((16,), jnp.float32)
    slope16 = jnp.full((16,), 0.01, jnp.float32)
    nvec = jnp.full((16,), N, jnp.int32)

    def chunk(i, row, col):
        idx = pl.ds(i * 16, 16)
        f = flat_v[idx]
        fp = prev_v[idx]
        dstv = lax.div(f, nvec)
        srcv = f - dstv * nvec
        ssrc = plsc.load_gather(s_v, [srcv])
        sdst = plsc.load_gather(s_v, [dstv])
        t = ssrc - sdst
        e = jnp.where(t >= zero16, t, slope16 * t)
        p = jnp.where(f != fp, jnp.exp(e - mvec), zero16)
        p_m[row, pl.ds(col * 16, 16)] = p
        src_m[row, pl.ds(col * 16, 16)] = srcv
        dst_m[row, pl.ds(col * 16, 16)] = dstv

    def rowbody(j, carry):
        for k in range(8):
            chunk(j * 8 + k, j, k)
        return carry

    lax.fori_loop(0, NROWS, rowbody, 0)

    def scatter_row(j, carry):
        pltpu.sync_copy(p_m.at[j], den_sh.at[dst_m.at[j]], add=True)
        return carry

    lax.fori_loop(0, NROWS, scatter_row, 0)
    pltpu.sync_copy(p_m, p_hbm.at[pl.ds(wid * NROWS, NROWS)])
    pltpu.sync_copy(src_m, src_hbm.at[pl.ds(wid * NROWS, NROWS)])
    pltpu.sync_copy(dst_m, dst_hbm.at[pl.ds(wid * NROWS, NROWS)])
    plsc.subcore_barrier()
    pltpu.sync_copy(den_sh.at[pl.ds(sid * STRIPE, STRIPE)],
                    den_hbm.at[pl.ds(c * ND + sid * STRIPE, STRIPE)])


_DNUMS = lax.GatherDimensionNumbers(
    offset_dims=(), collapsed_slice_dims=(0,), start_index_map=(0,))


def _make_sc_pass_b(F):
    @functools.partial(
        pl.kernel,
        out_type=jax.ShapeDtypeStruct((NC * ND, F), jnp.float32),
        mesh=_MESH,
        scratch_types=[
            pltpu.VMEM((NROWS, GRP), jnp.float32),
            pltpu.VMEM((NROWS, GRP), jnp.int32),
            pltpu.VMEM((NROWS, GRP), jnp.int32),
            pltpu.VMEM((GRP, F), jnp.float32),
            pltpu.VMEM((GRP, F), jnp.float32),
            pltpu.VMEM_SHARED((ND, F), jnp.float32),
            pltpu.SemaphoreType.DMA,
            pltpu.SemaphoreType.DMA,
        ],
        compiler_params=pltpu.CompilerParams(needs_layout_passes=False),
    )
    def sc_pass_b(p_hbm, src_hbm, dst_hbm, hd_hbm, zrows_hbm,
                  acc_hbm, a_m, src_m, dst_m, rows0, rows1, acc_sh,
                  gsem, ssem):
        c = lax.axis_index("c")
        sid = lax.axis_index("s")
        wid = c * NS + sid
        pltpu.sync_copy(p_hbm.at[pl.ds(wid * NROWS, NROWS)], a_m)
        pltpu.sync_copy(src_hbm.at[pl.ds(wid * NROWS, NROWS)], src_m)
        pltpu.sync_copy(dst_hbm.at[pl.ds(wid * NROWS, NROWS)], dst_m)
        pltpu.sync_copy(zrows_hbm, acc_sh.at[pl.ds(sid * STRIPE, STRIPE)])
        plsc.subcore_barrier()

        def issue_gather(g, buf):
            pltpu.async_copy(hd_hbm.at[src_m.at[g]], buf, gsem)

        def wait_gather(g, buf):
            pltpu.make_async_copy(hd_hbm.at[src_m.at[g]], buf, gsem).wait()

        def issue_scatter(g, buf):
            pltpu.async_copy(buf, acc_sh.at[dst_m.at[g]], ssem, add=True)

        def drain_scatter(buf):
            pltpu.make_async_copy(hd_hbm.at[pl.ds(0, GRP)], buf, ssem).wait()

        def scale(g, buf):
            def sub_body(sub, carry):
                a16 = a_m[g, pl.ds(sub * 16, 16)]
                for r in range(16):
                    ar = lax.gather(
                        a16, jnp.full((16, 1), r, jnp.int32), _DNUMS, (1,),
                        mode=lax.GatherScatterMode.PROMISE_IN_BOUNDS)
                    for cc in range(F // 16):
                        cs = pl.ds(cc * 16, 16)
                        buf[sub * 16 + r, cs] = buf[sub * 16 + r, cs] * ar
                return carry

            lax.fori_loop(0, GRP // 16, sub_body, 0)

        issue_gather(0, rows0)

        def pair(gg, carry):
            g0 = 2 * gg
            g1 = g0 + 1

            @pl.when(gg > 0)
            def _():
                drain_scatter(rows1)
            issue_gather(g1, rows1)
            wait_gather(g0, rows0)
            scale(g0, rows0)
            issue_scatter(g0, rows0)

            @pl.when(gg < NROWS // 2 - 1)
            def _():
                drain_scatter(rows0)
                issue_gather(g0 + 2, rows0)
            wait_gather(g1, rows1)
            scale(g1, rows1)
            issue_scatter(g1, rows1)
            return carry

        lax.fori_loop(0, NROWS // 2, pair, 0)
        drain_scatter(rows0)
        drain_scatter(rows1)
        plsc.subcore_barrier()
        pltpu.sync_copy(acc_sh.at[pl.ds(sid * STRIPE, STRIPE)],
                        acc_hbm.at[pl.ds(c * ND + sid * STRIPE, STRIPE)])

    return sc_pass_b


_sc_pass_b128 = _make_sc_pass_b(F_IN)


def kernel(x, edge_index, enc_fc_W, enc_diff_W, enc_att, att, dec_fc_W,
           dec_diff_W, dec_att):
    del att
    f32 = jnp.float32

    flat = jnp.sort(edge_index[1].astype(jnp.int32) * N
                    + edge_index[0].astype(jnp.int32))
    flat_p = jnp.concatenate(
        [flat, jnp.full((EPAD - E,), PAD_DST * N, jnp.int32)])
    flat_prev = jnp.concatenate([jnp.full((1,), -1, jnp.int32), flat_p[:-1]])
    z640 = jnp.zeros((STRIPE,), f32)
    zrows128 = jnp.zeros((STRIPE, F_IN), f32)

    wdiffT1 = jnp.concatenate(
        [enc_diff_W.T, jnp.zeros((F_IN, F_IN - F_HID), f32)], axis=1)
    att1 = jnp.concatenate(
        [enc_att, jnp.zeros((F_IN - F_HID, 1), f32)], axis=0)
    z1, hd1, s1, smx1, smn1 = _tc_input_layer(x, enc_fc_W.T, wdiffT1, att1)
    m1 = _lrelu(smx1[0, 0] - smn1[0, 0])
    m16_1 = jnp.full((16,), m1, f32)
    s1f = s1.reshape(-1)
    s1p = jnp.concatenate([s1f, jnp.broadcast_to(s1f[:1], (ND - N,))])
    den1, p1, src1, dst1 = _sc_pass_a(flat_p, flat_prev, s1p, m16_1, z640)
    acc1 = _sc_pass_b128(p1, src1, dst1, hd1, zrows128)
    d1a = den1[:N].reshape(N, 1)
    d1b = den1[ND:ND + N].reshape(N, 1)

    z2, hd2, s2, smx2, smn2 = _tc_mid_layer(
        z1, hd1, acc1.reshape(NC, ND, F_IN), d1a, d1b,
        dec_fc_W.T, dec_diff_W.T, dec_att)
    m2 = _lrelu(smx2[0, 0] - smn2[0, 0])
    m16_2 = jnp.full((16,), m2, f32)
    s2f = s2.reshape(-1)
    s2p = jnp.concatenate([s2f, jnp.broadcast_to(s2f[:1], (ND - N,))])
    den2, p2, src2, dst2 = _sc_pass_a(flat_p, flat_prev, s2p, m16_2, z640)
    acc2 = _sc_pass_b128(p2, src2, dst2, hd2, zrows128)
    d2a = den2[:N].reshape(N, 1)
    d2b = den2[ND:ND + N].reshape(N, 1)

    out = _tc_out_layer(z2, hd2, acc2.reshape(NC, ND, F_IN), d2a, d2b)
    return out
